# baseline (speedup 1.0000x reference)
import jax
import jax.numpy as jnp
from jax import lax
from jax.experimental import pallas as pl
from jax.experimental.pallas import tpu as pltpu
from jax.experimental.pallas import tpu_sc as plsc

_N = 16384 * 128
_NW = 32
_PER_W = _N // _NW
_CHUNK = 4096
_NCHUNK = _PER_W // _CHUNK
_D = 3
_L = 16
_SEQ = 1000000
_SEQ_PART = 62496
_STAGE = 4096


def _bsgen_body(src_hbm, seq_hbm, idx_hbm, out_hbm, seq_sh, *rest):
    idx = rest[0:_D]
    gat = rest[_D:2 * _D]
    src = rest[2 * _D:3 * _D]
    out = rest[3 * _D:4 * _D]
    si = rest[4 * _D:5 * _D]
    sg = rest[5 * _D:6 * _D]
    ss = rest[6 * _D:7 * _D]
    so = rest[7 * _D:8 * _D]
    st = rest[8 * _D:8 * _D + 4]

    sid = lax.axis_index("s")
    wid = sid * 2 + lax.axis_index("c")
    base = wid * _PER_W

    def fire_idx(c):
        b = c % _D
        return pltpu.async_copy(
            idx_hbm.at[pl.ds(base + c * _CHUNK, _CHUNK)], idx[b], si[b])

    def fire_src(c):
        b = c % _D
        return pltpu.async_copy(
            src_hbm.at[pl.ds(base + c * _CHUNK, _CHUNK)], src[b], ss[b])

    icp = {c: fire_idx(c) for c in range(_D)}
    scp = {c: fire_src(c) for c in range(_D)}

    sbase = sid * _SEQ_PART
    sizes = [_STAGE] * (_SEQ_PART // _STAGE) + [_SEQ_PART % _STAGE]
    nst = len(sizes)
    sbufs = (gat[0], gat[1], out[0], out[1])
    sosems = (sg[0], sg[1], so[0], so[1])

    def fire_stage_in(k):
        b = k % 4
        return pltpu.async_copy(
            seq_hbm.at[pl.ds(sbase + k * _STAGE, sizes[k])],
            sbufs[b].at[pl.ds(0, sizes[k])], st[b])

    def fire_stage_out(k):
        b = k % 4
        return pltpu.async_copy(
            sbufs[b].at[pl.ds(0, sizes[k])],
            seq_sh.at[pl.ds(sbase + k * _STAGE, sizes[k])], sosems[b])

    sin = {k: fire_stage_in(k) for k in range(min(4, nst))}
    sout = {}
    for k in range(nst):
        sin.pop(k).wait()
        sout[k] = fire_stage_out(k)
        if k + 4 < nst:
            sout.pop(k).wait()
            sin[k + 4] = fire_stage_in(k + 4)
    for k in sorted(sout):
        sout.pop(k).wait()

    @pl.when(sid == 15)
    def _copy_tail():
        tail = 16 * _SEQ_PART
        pltpu.sync_copy(seq_hbm.at[pl.ds(tail, _SEQ - tail)],
                        gat[0].at[pl.ds(0, _SEQ - tail)])
        pltpu.sync_copy(gat[0].at[pl.ds(0, _SEQ - tail)],
                        seq_sh.at[pl.ds(tail, _SEQ - tail)])

    plsc.subcore_barrier()

    def fire_gather(c):
        b = c % _D
        return pltpu.async_copy(seq_sh.at[idx[b]], gat[b], sg[b])

    icp.pop(0).wait()
    gcp = {0: fire_gather(0)}
    ocp = {}

    for c in range(_NCHUNK):
        b = c % _D
        if c + 1 < _NCHUNK:
            icp.pop(c + 1).wait()
            gcp[c + 1] = fire_gather(c + 1)
        gcp.pop(c).wait()
        scp.pop(c).wait()
        if c - (_D - 1) >= 0:
            ocp.pop(c - (_D - 1)).wait()

        def cmp_body(i, b=b):
            sv = src[b][pl.ds(i, _L)]
            gv = gat[b][pl.ds(i, _L)]
            out[b][pl.ds(i, _L)] = jnp.where(
                sv > gv, jnp.float32(1.0), jnp.float32(0.0))

        plsc.parallel_loop(0, _CHUNK, _L, unroll=16)(cmp_body)
        ocp[c] = pltpu.async_copy(
            out[b], out_hbm.at[pl.ds(base + c * _CHUNK, _CHUNK)], so[b])
        if c + _D < _NCHUNK:
            icp[c + _D] = fire_idx(c + _D)
            scp[c + _D] = fire_src(c + _D)

    for c in sorted(ocp):
        ocp.pop(c).wait()


@jax.jit
def _bsgen(src, seq, idx):
    mesh = plsc.VectorSubcoreMesh(core_axis_name="c", subcore_axis_name="s")
    scratch = [pltpu.VMEM_SHARED((_SEQ,), jnp.float32)]
    scratch += [pltpu.VMEM((_CHUNK,), jnp.int32) for _ in range(_D)]
    scratch += [pltpu.VMEM((_CHUNK,), jnp.float32) for _ in range(3 * _D)]
    scratch += [pltpu.SemaphoreType.DMA for _ in range(4 * _D + 4)]
    return pl.kernel(
        _bsgen_body,
        out_type=jax.ShapeDtypeStruct((_N,), jnp.float32),
        mesh=mesh,
        scratch_types=scratch,
    )(src, seq, idx)


def kernel(source, rng_seq, rng_idx):
    idx = rng_idx.astype(jnp.int32).reshape(_N)
    src = source.reshape(_N)
    out = _bsgen(src, rng_seq, idx)
    return out.reshape(source.shape)


# speedup vs baseline: 1.0033x; 1.0033x over previous
import jax
import jax.numpy as jnp
from jax import lax
from jax.experimental import pallas as pl
from jax.experimental.pallas import tpu as pltpu
from jax.experimental.pallas import tpu_sc as plsc

_N = 16384 * 128
_NW = 32
_PER_W = _N // _NW
_CHUNK = 8192
_NCHUNK = _PER_W // _CHUNK
_L = 16
_SEQ = 1000000
_SEQ_PART = 62496
_STAGE = 8192


def _bsgen_body(src_hbm, seq_hbm, idx_hbm, out_hbm, seq_sh, idx0, idx1, gat0,
                gat1, src0, src1, out0, out1, si0, si1, sg0, sg1, ss0, ss1,
                so0, so1, st0, st1, st2, st3):
    sid = lax.axis_index("s")
    wid = sid * 2 + lax.axis_index("c")
    base = wid * _PER_W

    idx = (idx0, idx1)
    gat = (gat0, gat1)
    src = (src0, src1)
    out = (out0, out1)
    si = (si0, si1)
    sg = (sg0, sg1)
    ss = (ss0, ss1)
    so = (so0, so1)
    st_in = (st0, st1)
    st_out = (st2, st3)

    def fire_idx(c):
        b = c & 1
        return pltpu.async_copy(
            idx_hbm.at[pl.ds(base + c * _CHUNK, _CHUNK)], idx[b], si[b])

    def fire_src(c):
        b = c & 1
        return pltpu.async_copy(
            src_hbm.at[pl.ds(base + c * _CHUNK, _CHUNK)], src[b], ss[b])

    icp = {0: fire_idx(0), 1: fire_idx(1)}
    scp = {0: fire_src(0), 1: fire_src(1)}

    sbase = sid * _SEQ_PART
    sizes = [_STAGE] * (_SEQ_PART // _STAGE) + [_SEQ_PART % _STAGE]
    nst = len(sizes)
    sbufs = (gat0, gat1, out0, out1)
    sisems = (st0, st1, st2, st3)
    sosems = (sg0, sg1, so0, so1)

    def fire_stage_in(k):
        b = k % 4
        return pltpu.async_copy(
            seq_hbm.at[pl.ds(sbase + k * _STAGE, sizes[k])],
            sbufs[b].at[pl.ds(0, sizes[k])], sisems[b])

    def fire_stage_out(k):
        b = k % 4
        return pltpu.async_copy(
            sbufs[b].at[pl.ds(0, sizes[k])],
            seq_sh.at[pl.ds(sbase + k * _STAGE, sizes[k])], sosems[b])

    sin = {k: fire_stage_in(k) for k in range(min(4, nst))}
    sout = {}
    for k in range(nst):
        sin.pop(k).wait()
        sout[k] = fire_stage_out(k)
        if k + 4 < nst:
            sout.pop(k).wait()
            sin[k + 4] = fire_stage_in(k + 4)
    for k in sorted(sout):
        sout.pop(k).wait()

    @pl.when(sid == 15)
    def _copy_tail():
        tail = 16 * _SEQ_PART
        pltpu.sync_copy(seq_hbm.at[pl.ds(tail, _SEQ - tail)],
                        gat0.at[pl.ds(0, _SEQ - tail)])
        pltpu.sync_copy(gat0.at[pl.ds(0, _SEQ - tail)],
                        seq_sh.at[pl.ds(tail, _SEQ - tail)])

    plsc.subcore_barrier()

    def fire_gather(c):
        b = c & 1
        return pltpu.async_copy(seq_sh.at[idx[b]], gat[b], sg[b])

    icp.pop(0).wait()
    gcp = {0: fire_gather(0)}
    ocp = {}

    for c in range(_NCHUNK):
        b = c & 1
        if c + 1 < _NCHUNK:
            icp.pop(c + 1).wait()
            gcp[c + 1] = fire_gather(c + 1)
        gcp.pop(c).wait()
        scp.pop(c).wait()
        if c - 2 >= 0:
            ocp.pop(c - 2).wait()

        def cmp_body(i, b=b):
            sv = src[b][pl.ds(i, _L)]
            gv = gat[b][pl.ds(i, _L)]
            out[b][pl.ds(i, _L)] = jnp.where(
                sv > gv, jnp.float32(1.0), jnp.float32(0.0))

        plsc.parallel_loop(0, _CHUNK, _L, unroll=8)(cmp_body)
        ocp[c] = pltpu.async_copy(
            out[b], out_hbm.at[pl.ds(base + c * _CHUNK, _CHUNK)], so[b])
        if c + 2 < _NCHUNK:
            icp[c + 2] = fire_idx(c + 2)
            scp[c + 2] = fire_src(c + 2)

    ocp.pop(_NCHUNK - 2).wait()
    ocp.pop(_NCHUNK - 1).wait()


@jax.jit
def _bsgen(src, seq, idx):
    mesh = plsc.VectorSubcoreMesh(core_axis_name="c", subcore_axis_name="s")
    return pl.kernel(
        _bsgen_body,
        out_type=jax.ShapeDtypeStruct((_N,), jnp.float32),
        mesh=mesh,
        scratch_types=[
            pltpu.VMEM_SHARED((_SEQ,), jnp.float32),
            pltpu.VMEM((_CHUNK,), jnp.int32),
            pltpu.VMEM((_CHUNK,), jnp.int32),
            pltpu.VMEM((_CHUNK,), jnp.float32),
            pltpu.VMEM((_CHUNK,), jnp.float32),
            pltpu.VMEM((_CHUNK,), jnp.float32),
            pltpu.VMEM((_CHUNK,), jnp.float32),
            pltpu.VMEM((_CHUNK,), jnp.float32),
            pltpu.VMEM((_CHUNK,), jnp.float32),
            pltpu.SemaphoreType.DMA,
            pltpu.SemaphoreType.DMA,
            pltpu.SemaphoreType.DMA,
            pltpu.SemaphoreType.DMA,
            pltpu.SemaphoreType.DMA,
            pltpu.SemaphoreType.DMA,
            pltpu.SemaphoreType.DMA,
            pltpu.SemaphoreType.DMA,
            pltpu.SemaphoreType.DMA,
            pltpu.SemaphoreType.DMA,
            pltpu.SemaphoreType.DMA,
            pltpu.SemaphoreType.DMA,
        ],
    )(src, seq, idx)


def kernel(source, rng_seq, rng_idx):
    idx = rng_idx.astype(jnp.int32).reshape(_N)
    src = source.reshape(_N)
    out = _bsgen(src, rng_seq, idx)
    return out.reshape(source.shape)
